# baseline (speedup 1.0000x reference)
import functools

import jax
import jax.numpy as jnp
from jax.experimental import pallas as pl
from jax.experimental.pallas import tpu as pltpu


def _fused_kernel(f1_ref, f2_ref, w1_ref, w2_ref, b1_ref, b2_ref, o_ref,
                  *, H: int, G: int):
    W = G * H
    f32 = jnp.float32

    w_eff = jnp.dot(w1_ref[...].astype(f32), w2_ref[...].astype(f32),
                    preferred_element_type=f32)
    b_eff = jnp.dot(b1_ref[...].astype(f32), w2_ref[...].astype(f32),
                    preferred_element_type=f32) + b2_ref[0, 0]

    wa_t = jnp.concatenate([w_eff[:H]] * G, axis=0) if G > 1 else w_eff[:H]
    wb_t = jnp.concatenate([w_eff[H:]] * G, axis=0) if G > 1 else w_eff[H:]
    row_grp = jax.lax.broadcasted_iota(jnp.int32, (W, G), 0) // H
    col_grp = jax.lax.broadcasted_iota(jnp.int32, (W, G), 1)
    diag = row_grp == col_grp
    sa = jnp.where(diag, wa_t, 0.0)
    sb = jnp.where(diag, wb_t, 0.0)

    acc = jnp.dot(f1_ref[...], sa, preferred_element_type=f32)
    acc = acc + jnp.dot(f2_ref[...], sb, preferred_element_type=f32)
    o_ref[...] = (acc + b_eff).astype(o_ref.dtype)


@functools.partial(jax.jit, static_argnames=("tm",))
def _mix_fusion(feat1, feat2, w1, b1, w2, b2, tm=4096):
    B, T, H = feat1.shape
    M = B * T
    out_dtype = feat1.dtype

    if H < 128 and 128 % H == 0 and M % (128 // H) == 0:
        G = 128 // H
    else:
        G = 1
    W = G * H
    Mf = M // G

    f1 = feat1.reshape(Mf, W)
    f2 = feat2.reshape(Mf, W)
    b1r = b1.reshape(1, H)
    b2r = b2.reshape(1, 1)

    tm = min(tm, Mf)
    grid = (pl.cdiv(Mf, tm),)

    out = pl.pallas_call(
        functools.partial(_fused_kernel, H=H, G=G),
        out_shape=jax.ShapeDtypeStruct((Mf, G), out_dtype),
        grid=grid,
        in_specs=[
            pl.BlockSpec((tm, W), lambda i: (i, 0)),
            pl.BlockSpec((tm, W), lambda i: (i, 0)),
            pl.BlockSpec((2 * H, H), lambda i: (0, 0)),
            pl.BlockSpec((H, 1), lambda i: (0, 0)),
            pl.BlockSpec((1, H), lambda i: (0, 0)),
            pl.BlockSpec(memory_space=pltpu.MemorySpace.SMEM),
        ],
        out_specs=pl.BlockSpec((tm, G), lambda i: (i, 0)),
        compiler_params=pltpu.CompilerParams(
            dimension_semantics=("parallel",),
        ),
    )(f1, f2, w1, w2, b1r, b2r)

    return jnp.squeeze(out.reshape(B, T, 1))


def kernel(feat1, feat2, score1, score2, w1, b1, w2, b2):
    del score1, score2
    return _mix_fusion(feat1, feat2, w1, b1, w2, b2)


# speedup vs baseline: 1.5957x; 1.5957x over previous
import functools

import jax
import jax.numpy as jnp
from jax.experimental import pallas as pl
from jax.experimental.pallas import tpu as pltpu


def _fused_kernel(f1_ref, f2_ref, w1_ref, w2_ref, b1_ref, b2_ref, o_ref,
                  *, H: int):
    f32 = jnp.float32

    w_eff = jnp.dot(w1_ref[...].astype(f32), w2_ref[...].astype(f32),
                    preferred_element_type=f32)
    b_eff = jnp.dot(b1_ref[...].astype(f32), w2_ref[...].astype(f32),
                    preferred_element_type=f32) + b2_ref[0, 0]
    wa = jax.lax.transpose(w_eff[:H], (1, 0)).reshape(1, 1, H)
    wb = jax.lax.transpose(w_eff[H:], (1, 0)).reshape(1, 1, H)

    s = f1_ref[...] * wa + f2_ref[...] * wb
    o = jnp.sum(s, axis=-1)
    o_ref[...] = (o + b_eff).astype(o_ref.dtype)


@functools.partial(jax.jit, static_argnames=("bm",))
def _mix_fusion(feat1, feat2, w1, b1, w2, b2, bm=64):
    B, T, H = feat1.shape
    out_dtype = feat1.dtype

    b1r = b1.reshape(1, H)
    b2r = b2.reshape(1, 1)

    bm = min(bm, B)
    grid = (pl.cdiv(B, bm),)

    out = pl.pallas_call(
        functools.partial(_fused_kernel, H=H),
        out_shape=jax.ShapeDtypeStruct((B, T), out_dtype),
        grid=grid,
        in_specs=[
            pl.BlockSpec((bm, T, H), lambda i: (i, 0, 0)),
            pl.BlockSpec((bm, T, H), lambda i: (i, 0, 0)),
            pl.BlockSpec((2 * H, H), lambda i: (0, 0)),
            pl.BlockSpec((H, 1), lambda i: (0, 0)),
            pl.BlockSpec((1, H), lambda i: (0, 0)),
            pl.BlockSpec(memory_space=pltpu.MemorySpace.SMEM),
        ],
        out_specs=pl.BlockSpec((bm, T), lambda i: (i, 0)),
        compiler_params=pltpu.CompilerParams(
            dimension_semantics=("parallel",),
        ),
    )(feat1, feat2, w1, w2, b1r, b2r)

    return out


def kernel(feat1, feat2, score1, score2, w1, b1, w2, b2):
    del score1, score2
    return _mix_fusion(feat1, feat2, w1, b1, w2, b2)


# speedup vs baseline: 7.9078x; 4.9557x over previous
import functools

import jax
import jax.numpy as jnp
from jax.experimental import pallas as pl
from jax.experimental.pallas import tpu as pltpu


def _fused_kernel(f1_ref, f2_ref, w1_ref, w2_ref, b1_ref, b2_ref, o_ref,
                  *, H: int):
    f32 = jnp.float32

    w_eff = jnp.dot(w1_ref[...].astype(f32), w2_ref[...].astype(f32),
                    preferred_element_type=f32)
    b_eff = jnp.dot(b1_ref[...].astype(f32), w2_ref[...].astype(f32),
                    preferred_element_type=f32) + b2_ref[0, 0]
    wa = w_eff[:H].reshape(1, H, 1)
    wb = w_eff[H:].reshape(1, H, 1)

    s = f1_ref[...] * wa + f2_ref[...] * wb
    o = jnp.sum(s, axis=1)
    o_ref[...] = (o + b_eff).astype(o_ref.dtype)


@functools.partial(jax.jit, static_argnames=("bm",))
def _mix_fusion(feat1, feat2, w1, b1, w2, b2, bm=64):
    B, T, H = feat1.shape
    out_dtype = feat1.dtype

    f1t = jnp.transpose(feat1, (0, 2, 1))
    f2t = jnp.transpose(feat2, (0, 2, 1))
    b1r = b1.reshape(1, H)
    b2r = b2.reshape(1, 1)

    bm = min(bm, B)
    grid = (pl.cdiv(B, bm),)

    out = pl.pallas_call(
        functools.partial(_fused_kernel, H=H),
        out_shape=jax.ShapeDtypeStruct((B, T), out_dtype),
        grid=grid,
        in_specs=[
            pl.BlockSpec((bm, H, T), lambda i: (i, 0, 0)),
            pl.BlockSpec((bm, H, T), lambda i: (i, 0, 0)),
            pl.BlockSpec((2 * H, H), lambda i: (0, 0)),
            pl.BlockSpec((H, 1), lambda i: (0, 0)),
            pl.BlockSpec((1, H), lambda i: (0, 0)),
            pl.BlockSpec(memory_space=pltpu.MemorySpace.SMEM),
        ],
        out_specs=pl.BlockSpec((bm, T), lambda i: (i, 0)),
        compiler_params=pltpu.CompilerParams(
            dimension_semantics=("parallel",),
        ),
    )(f1t, f2t, w1, w2, b1r, b2r)

    return out


def kernel(feat1, feat2, score1, score2, w1, b1, w2, b2):
    del score1, score2
    return _mix_fusion(feat1, feat2, w1, b1, w2, b2)
